# Initial kernel scaffold; baseline (speedup 1.0000x reference)
import jax
import jax.numpy as jnp
from jax.experimental import pallas as pl


def kernel(char_indices, embedding_weight):
    raise NotImplementedError("write your pallas kernel here")


# speedup vs baseline: 3.5019x; 3.5019x over previous
import math
import functools

import jax
import jax.numpy as jnp
from jax import lax
from jax.experimental import pallas as pl
from jax.experimental.pallas import tpu as pltpu
from jax.experimental.pallas import tpu_sc as plsc

VOCAB = 100000
DIM = 64
BATCH = 4096
SEQ = 200

_info = plsc.get_sparse_core_info()
NC, NS = _info.num_cores, _info.num_subcores
NW = NC * NS
ROWS_PER_W = BATCH * SEQ // NW
HALF = 100
SEQ_PER_W = BATCH // NW


def _pos_encoding():
    position = jnp.arange(SEQ, dtype=jnp.float32)[:, None]
    div_term = jnp.exp(
        jnp.arange(0, DIM, 2, dtype=jnp.float32) * (-math.log(10000.0) / DIM)
    )
    pe = jnp.zeros((SEQ, DIM), dtype=jnp.float32)
    pe = pe.at[:, 0::2].set(jnp.sin(position * div_term))
    pe = pe.at[:, 1::2].set(jnp.cos(position * div_term))
    return pe


def _body(idx_hbm, table_hbm, pe_hbm, out_hbm, idx_v, pe_sh, buf, sem):
    c = lax.axis_index("c")
    s = lax.axis_index("s")
    wid = s * NC + c
    pltpu.sync_copy(idx_hbm.at[wid], idx_v)

    @pl.when(s == 0)
    def _():
        pltpu.sync_copy(pe_hbm, pe_sh)

    plsc.subcore_barrier()

    def chunk(j, carry):
        pltpu.sync_copy(pe_sh, buf)
        a = pltpu.async_copy(
            table_hbm.at[idx_v.at[2 * j]], buf.at[pl.ds(0, HALF)], sem, add=True
        )
        b = pltpu.async_copy(
            table_hbm.at[idx_v.at[2 * j + 1]],
            buf.at[pl.ds(HALF, HALF)],
            sem,
            add=True,
        )
        a.wait()
        b.wait()
        row0 = wid * ROWS_PER_W + j * SEQ
        pltpu.sync_copy(buf, out_hbm.at[pl.ds(row0, SEQ)])
        return carry

    lax.fori_loop(0, SEQ_PER_W, chunk, 0)


@jax.jit
def _run(idx, table, pe):
    mesh = plsc.VectorSubcoreMesh(core_axis_name="c", subcore_axis_name="s")
    k = pl.kernel(
        _body,
        out_type=jax.ShapeDtypeStruct((BATCH * SEQ, DIM), jnp.float32),
        mesh=mesh,
        compiler_params=pltpu.CompilerParams(use_tc_tiling_on_sc=False),
        scratch_types=[
            pltpu.VMEM((2 * SEQ_PER_W, HALF), jnp.int32),
            pltpu.VMEM_SHARED((SEQ, DIM), jnp.float32),
            pltpu.VMEM((SEQ, DIM), jnp.float32),
            pltpu.SemaphoreType.DMA,
        ],
    )
    return k(idx, table, pe)


def kernel(char_indices, embedding_weight):
    idx = char_indices.astype(jnp.int32).reshape(NW, 2 * SEQ_PER_W, HALF)
    pe = _pos_encoding()
    out = _run(idx, embedding_weight, pe)
    return out.reshape(BATCH, SEQ, DIM)


# speedup vs baseline: 4.1951x; 1.1979x over previous
import math
import functools

import jax
import jax.numpy as jnp
from jax import lax
from jax.experimental import pallas as pl
from jax.experimental.pallas import tpu as pltpu
from jax.experimental.pallas import tpu_sc as plsc

VOCAB = 100000
DIM = 64
BATCH = 4096
SEQ = 200

_info = plsc.get_sparse_core_info()
NC, NS = _info.num_cores, _info.num_subcores
NW = NC * NS
ROWS_PER_W = BATCH * SEQ // NW
HALF = 100
SEQ_PER_W = BATCH // NW


def _pos_encoding():
    position = jnp.arange(SEQ, dtype=jnp.float32)[:, None]
    div_term = jnp.exp(
        jnp.arange(0, DIM, 2, dtype=jnp.float32) * (-math.log(10000.0) / DIM)
    )
    pe = jnp.zeros((SEQ, DIM), dtype=jnp.float32)
    pe = pe.at[:, 0::2].set(jnp.sin(position * div_term))
    pe = pe.at[:, 1::2].set(jnp.cos(position * div_term))
    return pe


NBUF = 4


def _body(idx_hbm, table_hbm, pe_hbm, out_hbm, idx_v, pe_sh, bufs, sem_g, sem_o):
    c = lax.axis_index("c")
    s = lax.axis_index("s")
    wid = s * NC + c
    pltpu.sync_copy(idx_hbm.at[wid], idx_v)

    @pl.when(s == 0)
    def _():
        pltpu.sync_copy(pe_hbm, pe_sh)

    plsc.subcore_barrier()

    def gather_descs(j, b):
        ga = pltpu.make_async_copy(
            table_hbm.at[idx_v.at[2 * j]], bufs.at[b, pl.ds(0, HALF)], sem_g.at[b]
        )
        gb = pltpu.make_async_copy(
            table_hbm.at[idx_v.at[2 * j + 1]],
            bufs.at[b, pl.ds(HALF, HALF)],
            sem_g.at[b],
        )
        return ga, gb

    def out_desc(j, b):
        row0 = wid * ROWS_PER_W + j * SEQ
        return pltpu.make_async_copy(
            bufs.at[b], out_hbm.at[pl.ds(row0, SEQ)], sem_o.at[b]
        )

    def prep(j):
        b = j % NBUF
        pltpu.sync_copy(pe_sh, bufs.at[b])
        pltpu.async_copy(
            table_hbm.at[idx_v.at[2 * j]],
            bufs.at[b, pl.ds(0, HALF)],
            sem_g.at[b],
            add=True,
        )
        pltpu.async_copy(
            table_hbm.at[idx_v.at[2 * j + 1]],
            bufs.at[b, pl.ds(HALF, HALF)],
            sem_g.at[b],
            add=True,
        )

    prep(0)
    prep(1)

    def step(j, carry):
        b = j % NBUF
        ga, gb = gather_descs(j, b)
        ga.wait()
        gb.wait()
        out_desc(j, b).start()

        @pl.when(j + 2 < SEQ_PER_W)
        def _():
            bn = (j + 2) % NBUF

            @pl.when(j >= 2)
            def _():
                out_desc(j - 2, bn).wait()

            prep(j + 2)

        return carry

    lax.fori_loop(0, SEQ_PER_W, step, 0)

    for t in range(NBUF):
        jj = SEQ_PER_W - NBUF + t
        out_desc(jj, jj % NBUF).wait()


@jax.jit
def _run(idx, table, pe):
    mesh = plsc.VectorSubcoreMesh(core_axis_name="c", subcore_axis_name="s")
    k = pl.kernel(
        _body,
        out_type=jax.ShapeDtypeStruct((BATCH * SEQ, DIM), jnp.float32),
        mesh=mesh,
        compiler_params=pltpu.CompilerParams(use_tc_tiling_on_sc=False),
        scratch_types=[
            pltpu.VMEM((2 * SEQ_PER_W, HALF), jnp.int32),
            pltpu.VMEM_SHARED((SEQ, DIM), jnp.float32),
            pltpu.VMEM((NBUF, SEQ, DIM), jnp.float32),
            pltpu.SemaphoreType.DMA((NBUF,)),
            pltpu.SemaphoreType.DMA((NBUF,)),
        ],
    )
    return k(idx, table, pe)


def kernel(char_indices, embedding_weight):
    idx = char_indices.astype(jnp.int32).reshape(NW, 2 * SEQ_PER_W, HALF)
    pe = _pos_encoding()
    out = _run(idx, embedding_weight, pe)
    return out.reshape(BATCH, SEQ, DIM)


# speedup vs baseline: 4.1960x; 1.0002x over previous
import math
import functools

import jax
import jax.numpy as jnp
from jax import lax
from jax.experimental import pallas as pl
from jax.experimental.pallas import tpu as pltpu
from jax.experimental.pallas import tpu_sc as plsc

VOCAB = 100000
DIM = 64
BATCH = 4096
SEQ = 200

_info = plsc.get_sparse_core_info()
NC, NS = _info.num_cores, _info.num_subcores
NW = NC * NS
ROWS_PER_W = BATCH * SEQ // NW
HALF = 100
SEQ_PER_W = BATCH // NW


def _pos_encoding():
    position = jnp.arange(SEQ, dtype=jnp.float32)[:, None]
    div_term = jnp.exp(
        jnp.arange(0, DIM, 2, dtype=jnp.float32) * (-math.log(10000.0) / DIM)
    )
    pe = jnp.zeros((SEQ, DIM), dtype=jnp.float32)
    pe = pe.at[:, 0::2].set(jnp.sin(position * div_term))
    pe = pe.at[:, 1::2].set(jnp.cos(position * div_term))
    return pe


NBUF = 4


def _body(idx_hbm, table_hbm, pe_hbm, out_hbm, idx_v, pe_sh, bufs, sem_g, sem_o):
    c = lax.axis_index("c")
    s = lax.axis_index("s")
    wid = s * NC + c
    pltpu.sync_copy(idx_hbm.at[wid], idx_v)

    @pl.when(s == 0)
    def _():
        pltpu.sync_copy(pe_hbm, pe_sh)

    plsc.subcore_barrier()

    def gather_descs(j, b):
        ga = pltpu.make_async_copy(
            table_hbm.at[idx_v.at[2 * j]], bufs.at[b, pl.ds(0, HALF)], sem_g.at[b]
        )
        gb = pltpu.make_async_copy(
            table_hbm.at[idx_v.at[2 * j + 1]],
            bufs.at[b, pl.ds(HALF, HALF)],
            sem_g.at[b],
        )
        return ga, gb

    def out_desc(j, b):
        gseq = wid * SEQ_PER_W + j
        return pltpu.make_async_copy(bufs.at[b], out_hbm.at[gseq], sem_o.at[b])

    def prep(j):
        b = j % NBUF
        pltpu.sync_copy(pe_sh, bufs.at[b])
        pltpu.async_copy(
            table_hbm.at[idx_v.at[2 * j]],
            bufs.at[b, pl.ds(0, HALF)],
            sem_g.at[b],
            add=True,
        )
        pltpu.async_copy(
            table_hbm.at[idx_v.at[2 * j + 1]],
            bufs.at[b, pl.ds(HALF, HALF)],
            sem_g.at[b],
            add=True,
        )

    prep(0)
    prep(1)

    def step(j, carry):
        b = j % NBUF
        ga, gb = gather_descs(j, b)
        ga.wait()
        gb.wait()
        out_desc(j, b).start()

        @pl.when(j + 2 < SEQ_PER_W)
        def _():
            bn = (j + 2) % NBUF

            @pl.when(j >= 2)
            def _():
                out_desc(j - 2, bn).wait()

            prep(j + 2)

        return carry

    lax.fori_loop(0, SEQ_PER_W, step, 0)

    for t in range(NBUF):
        jj = SEQ_PER_W - NBUF + t
        out_desc(jj, jj % NBUF).wait()


@jax.jit
def _run(idx, table, pe):
    mesh = plsc.VectorSubcoreMesh(core_axis_name="c", subcore_axis_name="s")
    k = pl.kernel(
        _body,
        out_type=jax.ShapeDtypeStruct((BATCH, SEQ, DIM), jnp.float32),
        mesh=mesh,
        compiler_params=pltpu.CompilerParams(use_tc_tiling_on_sc=False),
        scratch_types=[
            pltpu.VMEM((2 * SEQ_PER_W, HALF), jnp.int32),
            pltpu.VMEM_SHARED((SEQ, DIM), jnp.float32),
            pltpu.VMEM((NBUF, SEQ, DIM), jnp.float32),
            pltpu.SemaphoreType.DMA((NBUF,)),
            pltpu.SemaphoreType.DMA((NBUF,)),
        ],
    )
    return k(idx, table, pe)


def kernel(char_indices, embedding_weight):
    idx = char_indices.astype(jnp.int32).reshape(NW, 2 * SEQ_PER_W, HALF)
    pe = _pos_encoding()
    return _run(idx, embedding_weight, pe)
